# baseline (speedup 1.0000x reference)
import functools

import jax
import jax.numpy as jnp
from jax import lax
from jax.experimental import pallas as pl
from jax.experimental.pallas import tpu as pltpu
from jax.experimental.pallas import tpu_sc as plsc

_SC_PARAMS = pltpu.CompilerParams(
    needs_layout_passes=False, use_tc_tiling_on_sc=False
)

NC = 2
NS = 16
NW = NC * NS
L = 16
CH = 128


def _sc_gather(wg, out_idx, x16, R, S):
    rows_per = R // NW
    nch = R // CH

    mesh = plsc.VectorSubcoreMesh(core_axis_name="c", subcore_axis_name="s")

    @functools.partial(
        pl.kernel,
        out_type=(
            jax.ShapeDtypeStruct((R, R), jnp.float32),
            jax.ShapeDtypeStruct((R,), jnp.float32),
        ),
        mesh=mesh,
        compiler_params=_SC_PARAMS,
        scratch_types=[
            pltpu.VMEM((nch, CH), jnp.int32),
            pltpu.VMEM((R,), jnp.int32),
            pltpu.VMEM((rows_per, R), jnp.int32),
            pltpu.VMEM((2, R, 16), jnp.float32),
            pltpu.VMEM((rows_per, R), jnp.float32),
            pltpu.VMEM((R,), jnp.float32),
            pltpu.SemaphoreType.DMA,
            pltpu.SemaphoreType.DMA,
        ],
    )
    def k(w_hbm, oi_hbm, x_hbm, wsub_hbm, xsub_hbm,
          cp_v, oil_v, gi_v, grow_v, rows_v, xs_v, sem_a, sem_b):
        wid = lax.axis_index("s") * NC + lax.axis_index("c")
        row0 = wid * rows_per
        pltpu.sync_copy(oi_hbm, gi_v.at[0])
        for c in range(nch):
            for t in range(CH // L):
                raw = gi_v[0, pl.ds(c * CH + t * L, L)]
                cp_v[c, pl.ds(t * L, L)] = ((raw >> 7) << 6) + ((raw >> 4) & 7)
                oil_v[pl.ds(c * CH + t * L, L)] = raw & 15

        def build(kk, _):
            r = row0 + kk
            rp = ((r >> 3) << 15) + ((kk & 7) << 3)
            for c in range(nch):
                for t in range(CH // L):
                    gi_v[kk, pl.ds(c * CH + t * L, L)] = (
                        cp_v[c, pl.ds(t * L, L)] + rp
                    )
            return 0

        lax.fori_loop(0, rows_per, build, 0)

        lane = lax.iota(jnp.int32, L)

        def fire(kk, sl, sem):
            pltpu.async_copy(w_hbm.at[gi_v.at[kk]], grow_v.at[sl], sem)

        def drain(sl, sem):
            pltpu.make_async_copy(
                w_hbm.at[pl.ds(0, R)], grow_v.at[sl], sem
            ).wait()

        def extract(kk, sl):
            for t in range(R // L):
                vals = plsc.load_gather(
                    grow_v.at[sl], [t * L + lane, oil_v[pl.ds(t * L, L)]]
                )
                rows_v[kk, pl.ds(t * L, L)] = vals

        fire(0, 0, sem_a)

        def pipe(m, _):
            kk0 = 2 * m
            fire(kk0 + 1, 1, sem_b)
            drain(0, sem_a)
            extract(kk0, 0)

            @pl.when(kk0 + 2 < rows_per)
            def _():
                fire(kk0 + 2, 0, sem_a)

            drain(1, sem_b)
            extract(kk0 + 1, 1)
            return 0

        lax.fori_loop(0, rows_per // 2, pipe, 0)
        pltpu.sync_copy(rows_v, wsub_hbm.at[pl.ds(row0, rows_per)])

        @pl.when(wid == 0)
        def _():
            for c in range(nch):
                for t in range(CH // L):
                    cp = cp_v[c, pl.ds(t * L, L)]
                    gi_v[0, pl.ds(c * CH + t * L, L)] = (
                        ((cp >> 6) << 3) + (cp & 7)
                    )
            pltpu.async_copy(
                x_hbm.at[gi_v.at[0]], grow_v.at[0], sem_a
            ).wait()
            for t in range(R // L):
                vals = plsc.load_gather(
                    grow_v.at[0], [t * L + lane, oil_v[pl.ds(t * L, L)]]
                )
                xs_v[pl.ds(t * L, L)] = vals
            pltpu.sync_copy(xs_v, xsub_hbm)

    return k(wg, out_idx, x16)


def _tc_dense(weights, xt, x2d, cw):
    R, S = weights.shape
    ncols = S // cw
    MV = R // 2

    def body(w_ref, xt_ref, x2_ref, out_ref, pv_ref, pm_ref):
        i = pl.program_id(0)

        @pl.when(i == 0)
        def _():
            pv_ref[...] = jnp.zeros_like(pv_ref)
            pm_ref[...] = jnp.zeros_like(pm_ref)

        xrow = x2_ref[pl.ds(lax.rem(i, 8), 1), :]
        pv_ref[...] += w_ref[pl.ds(0, MV), :] * xrow
        p = jnp.dot(
            w_ref[pl.ds(MV, R - MV), :],
            xt_ref[...],
            preferred_element_type=jnp.float32,
        )
        sel = lax.broadcasted_iota(jnp.int32, (1, ncols), 1) == i
        pm_ref[...] += jnp.where(sel, p, 0.0)

        @pl.when(i == ncols - 1)
        def _():
            out_ref[pl.ds(0, MV), :] = jnp.sum(
                pv_ref[...], axis=1, keepdims=True
            )
            out_ref[pl.ds(MV, R - MV), :] = jnp.sum(
                pm_ref[...], axis=1, keepdims=True
            )

    return pl.pallas_call(
        body,
        grid=(ncols,),
        in_specs=[
            pl.BlockSpec((R, cw), lambda i: (0, i)),
            pl.BlockSpec((cw, ncols), lambda i: (0, 0)),
            pl.BlockSpec((8, cw), lambda i: (i // 8, 0)),
        ],
        out_specs=pl.BlockSpec((R, 1), lambda i: (0, 0)),
        out_shape=jax.ShapeDtypeStruct((R, 1), jnp.float32),
        scratch_shapes=[
            pltpu.VMEM((MV, cw), jnp.float32),
            pltpu.VMEM((R - MV, ncols), jnp.float32),
        ],
    )(weights, xt, x2d)


def _tc_iterate(baset, b2, wsubt, xs2, sg2, mi):
    R = wsubt.shape[0]

    def body(base_ref, b_ref, w_ref, xs_ref, sg_ref, mi_ref, out_ref):
        base = base_ref[...] + b_ref[...]
        xs = xs_ref[...]
        sg = sg_ref[...]
        dv = sg - xs
        w = w_ref[...]
        mi_v = mi_ref[0]

        def cond(carry):
            i, _, changed = carry
            return ((i == 0) | (i <= mi_v)) & (i < 20) & changed

        def it(carry):
            i, ef, _ = carry
            delta = ef * dv
            act = base + jnp.dot(delta, w, preferred_element_type=jnp.float32)
            fired = (act > 0.0).astype(jnp.float32)
            ef2 = jnp.maximum(ef, fired)
            return (i + 1, ef2, jnp.any(ef2 != ef))

        _, ef, _ = lax.while_loop(
            cond,
            it,
            (jnp.int32(0), jnp.zeros((1, R), jnp.float32), jnp.bool_(True)),
        )
        out_ref[...] = jnp.where(ef > 0.0, sg, xs)

    return pl.pallas_call(
        body,
        in_specs=[
            pl.BlockSpec(memory_space=pltpu.VMEM),
            pl.BlockSpec(memory_space=pltpu.VMEM),
            pl.BlockSpec(memory_space=pltpu.VMEM),
            pl.BlockSpec(memory_space=pltpu.VMEM),
            pl.BlockSpec(memory_space=pltpu.VMEM),
            pl.BlockSpec(memory_space=pltpu.SMEM),
        ],
        out_specs=pl.BlockSpec(memory_space=pltpu.VMEM),
        out_shape=jax.ShapeDtypeStruct((1, R), jnp.float32),
    )(baset, b2, wsubt, xs2, sg2, mi)


def _sc_scatter(x, out_idx, vals):
    S = x.shape[0]
    R = out_idx.shape[0]
    per = S // NW

    mesh = plsc.VectorSubcoreMesh(core_axis_name="c", subcore_axis_name="s")

    @functools.partial(
        pl.kernel,
        out_type=jax.ShapeDtypeStruct((S,), jnp.float32),
        mesh=mesh,
        compiler_params=_SC_PARAMS,
        scratch_types=[
            pltpu.VMEM((per,), jnp.float32),
            pltpu.VMEM((R,), jnp.int32),
            pltpu.VMEM((R,), jnp.float32),
        ],
    )
    def k(x_hbm, oi_hbm, val_hbm, out_hbm, xb_v, oi_v, val_v):
        wid = lax.axis_index("s") * NC + lax.axis_index("c")
        base = wid * per
        pltpu.sync_copy(x_hbm.at[pl.ds(base, per)], xb_v)
        pltpu.sync_copy(oi_hbm, oi_v)
        pltpu.sync_copy(val_hbm, val_v)
        for t in range(R // L):
            idx = oi_v[pl.ds(t * L, L)]
            v = val_v[pl.ds(t * L, L)]
            loc = idx - base
            m = (loc >= 0) & (loc < per)
            locc = jnp.clip(loc, 0, per - 1)
            plsc.store_scatter(xb_v, [locc], v, mask=m)
        pltpu.sync_copy(xb_v, out_hbm.at[pl.ds(base, per)])

    return k(x, out_idx, vals)


def kernel(x, weights, biases, out_idx, out_sign, max_iters):
    R, S = weights.shape
    wg = jnp.reshape(
        jnp.transpose(
            jnp.reshape(weights, (R // 8, 8, S // 128, 128)), (0, 2, 1, 3)
        ),
        (R * S // 16, 16),
    )
    x16 = jnp.reshape(x, (S // 16, 16))
    cw = 4096
    x2d = jnp.reshape(x, (S // cw, cw))
    base2 = _tc_dense(weights, jnp.transpose(x2d), x2d, cw)
    wsub, xsub = _sc_gather(wg, out_idx, x16, R, S)
    mi = jnp.reshape(jnp.asarray(max_iters, jnp.int32), (1,))
    vfin = _tc_iterate(
        jnp.reshape(base2, (1, R)),
        jnp.reshape(biases, (1, R)),
        jnp.transpose(wsub),
        jnp.reshape(xsub, (1, R)),
        jnp.reshape(out_sign, (1, R)),
        mi,
    )
    return _sc_scatter(x, out_idx, jnp.reshape(vfin, (R,)))


# speedup vs baseline: 1.0485x; 1.0485x over previous
import functools

import jax
import jax.numpy as jnp
from jax import lax
from jax.experimental import pallas as pl
from jax.experimental.pallas import tpu as pltpu
from jax.experimental.pallas import tpu_sc as plsc

_SC_PARAMS = pltpu.CompilerParams(
    needs_layout_passes=False, use_tc_tiling_on_sc=False
)

NC = 2
NS = 16
NW = NC * NS
L = 16
CH = 128


def _sc_gather(wg, out_idx, x16, R, S):
    rows_per = R // NW
    nch = R // CH

    mesh = plsc.VectorSubcoreMesh(core_axis_name="c", subcore_axis_name="s")

    @functools.partial(
        pl.kernel,
        out_type=(
            jax.ShapeDtypeStruct((R, R), jnp.float32),
            jax.ShapeDtypeStruct((R,), jnp.float32),
        ),
        mesh=mesh,
        compiler_params=_SC_PARAMS,
        scratch_types=[
            pltpu.VMEM((nch, CH), jnp.int32),
            pltpu.VMEM((R,), jnp.int32),
            pltpu.VMEM((rows_per, R), jnp.int32),
            pltpu.VMEM((2, R, 16), jnp.float32),
            pltpu.VMEM((rows_per, R), jnp.float32),
            pltpu.VMEM((R,), jnp.float32),
            pltpu.SemaphoreType.DMA,
            pltpu.SemaphoreType.DMA,
        ],
    )
    def k(w_hbm, oi_hbm, x_hbm, wsub_hbm, xsub_hbm,
          cp_v, oil_v, gi_v, grow_v, rows_v, xs_v, sem_a, sem_b):
        wid = lax.axis_index("s") * NC + lax.axis_index("c")
        row0 = wid * rows_per
        pltpu.sync_copy(oi_hbm, gi_v.at[0])
        for c in range(nch):
            for t in range(CH // L):
                raw = gi_v[0, pl.ds(c * CH + t * L, L)]
                cp_v[c, pl.ds(t * L, L)] = ((raw >> 7) << 6) + ((raw >> 4) & 7)
                oil_v[pl.ds(c * CH + t * L, L)] = raw & 15

        def build(kk, _):
            r = row0 + kk
            rp = ((r >> 3) << 15) + ((kk & 7) << 3)
            for c in range(nch):
                for t in range(CH // L):
                    gi_v[kk, pl.ds(c * CH + t * L, L)] = (
                        cp_v[c, pl.ds(t * L, L)] + rp
                    )
            return 0

        lax.fori_loop(0, rows_per, build, 0)

        lane = lax.iota(jnp.int32, L)

        def fire(kk, sl, sem):
            pltpu.async_copy(w_hbm.at[gi_v.at[kk]], grow_v.at[sl], sem)

        def drain(sl, sem):
            pltpu.make_async_copy(
                w_hbm.at[pl.ds(0, R)], grow_v.at[sl], sem
            ).wait()

        def extract(kk, sl):
            for t in range(R // L):
                vals = plsc.load_gather(
                    grow_v.at[sl], [t * L + lane, oil_v[pl.ds(t * L, L)]]
                )
                rows_v[kk, pl.ds(t * L, L)] = vals

        fire(0, 0, sem_a)

        def pipe(m, _):
            kk0 = 2 * m
            fire(kk0 + 1, 1, sem_b)
            drain(0, sem_a)
            extract(kk0, 0)

            @pl.when(kk0 + 2 < rows_per)
            def _():
                fire(kk0 + 2, 0, sem_a)

            drain(1, sem_b)
            extract(kk0 + 1, 1)
            return 0

        lax.fori_loop(0, rows_per // 2, pipe, 0)
        pltpu.sync_copy(rows_v, wsub_hbm.at[pl.ds(row0, rows_per)])

        @pl.when(wid == 0)
        def _():
            for c in range(nch):
                for t in range(CH // L):
                    cp = cp_v[c, pl.ds(t * L, L)]
                    gi_v[0, pl.ds(c * CH + t * L, L)] = (
                        ((cp >> 6) << 3) + (cp & 7)
                    )
            pltpu.async_copy(
                x_hbm.at[gi_v.at[0]], grow_v.at[0], sem_a
            ).wait()
            for t in range(R // L):
                vals = plsc.load_gather(
                    grow_v.at[0], [t * L + lane, oil_v[pl.ds(t * L, L)]]
                )
                xs_v[pl.ds(t * L, L)] = vals
            pltpu.sync_copy(xs_v, xsub_hbm)

    return k(wg, out_idx, x16)


def _tc_dense(weights, xt, x2d, cw):
    R, S = weights.shape
    ncols = S // cw
    MV = R // 2

    def body(w1_ref, w2_ref, xt_ref, x2_ref, out_ref, pv_ref, pm_ref):
        i = pl.program_id(0)

        @pl.when(i == 0)
        def _():
            pv_ref[...] = jnp.zeros_like(pv_ref)
            pm_ref[...] = jnp.zeros_like(pm_ref)

        xrow = x2_ref[pl.ds(lax.rem(i, 8), 1), :]
        pv_ref[...] += w1_ref[...] * xrow
        p = jnp.dot(
            w2_ref[...],
            xt_ref[...],
            preferred_element_type=jnp.float32,
        )
        sel = lax.broadcasted_iota(jnp.int32, (1, ncols), 1) == i
        pm_ref[...] += jnp.where(sel, p, 0.0)

        @pl.when(i == ncols - 1)
        def _():
            out_ref[pl.ds(0, MV), :] = jnp.sum(
                pv_ref[...], axis=1, keepdims=True
            )
            out_ref[pl.ds(MV, R - MV), :] = jnp.sum(
                pm_ref[...], axis=1, keepdims=True
            )

    return pl.pallas_call(
        body,
        grid=(ncols,),
        in_specs=[
            pl.BlockSpec((MV, cw), lambda i: (0, i)),
            pl.BlockSpec((R - MV, cw), lambda i: (1, i)),
            pl.BlockSpec((cw, ncols), lambda i: (0, 0)),
            pl.BlockSpec((8, cw), lambda i: (i // 8, 0)),
        ],
        out_specs=pl.BlockSpec((R, 1), lambda i: (0, 0)),
        out_shape=jax.ShapeDtypeStruct((R, 1), jnp.float32),
        scratch_shapes=[
            pltpu.VMEM((MV, cw), jnp.float32),
            pltpu.VMEM((R - MV, ncols), jnp.float32),
        ],
    )(weights, weights, xt, x2d)


def _tc_iterate(baset, b2, wsubt, xs2, sg2, mi):
    R = wsubt.shape[0]

    def body(base_ref, b_ref, w_ref, xs_ref, sg_ref, mi_ref, out_ref):
        base = base_ref[...] + b_ref[...]
        xs = xs_ref[...]
        sg = sg_ref[...]
        dv = sg - xs
        w = w_ref[...]
        mi_v = mi_ref[0]

        def cond(carry):
            i, _, changed = carry
            return ((i == 0) | (i <= mi_v)) & (i < 20) & changed

        def it(carry):
            i, ef, _ = carry
            delta = ef * dv
            act = base + jnp.dot(delta, w, preferred_element_type=jnp.float32)
            fired = (act > 0.0).astype(jnp.float32)
            ef2 = jnp.maximum(ef, fired)
            return (i + 1, ef2, jnp.any(ef2 != ef))

        _, ef, _ = lax.while_loop(
            cond,
            it,
            (jnp.int32(0), jnp.zeros((1, R), jnp.float32), jnp.bool_(True)),
        )
        out_ref[...] = jnp.where(ef > 0.0, sg, xs)

    return pl.pallas_call(
        body,
        in_specs=[
            pl.BlockSpec(memory_space=pltpu.VMEM),
            pl.BlockSpec(memory_space=pltpu.VMEM),
            pl.BlockSpec(memory_space=pltpu.VMEM),
            pl.BlockSpec(memory_space=pltpu.VMEM),
            pl.BlockSpec(memory_space=pltpu.VMEM),
            pl.BlockSpec(memory_space=pltpu.SMEM),
        ],
        out_specs=pl.BlockSpec(memory_space=pltpu.VMEM),
        out_shape=jax.ShapeDtypeStruct((1, R), jnp.float32),
    )(baset, b2, wsubt, xs2, sg2, mi)


def _sc_scatter(x, out_idx, vals):
    S = x.shape[0]
    R = out_idx.shape[0]
    per = S // NW

    mesh = plsc.VectorSubcoreMesh(core_axis_name="c", subcore_axis_name="s")

    @functools.partial(
        pl.kernel,
        out_type=jax.ShapeDtypeStruct((S,), jnp.float32),
        mesh=mesh,
        compiler_params=_SC_PARAMS,
        scratch_types=[
            pltpu.VMEM((per,), jnp.float32),
            pltpu.VMEM((R,), jnp.int32),
            pltpu.VMEM((R,), jnp.float32),
        ],
    )
    def k(x_hbm, oi_hbm, val_hbm, out_hbm, xb_v, oi_v, val_v):
        wid = lax.axis_index("s") * NC + lax.axis_index("c")
        base = wid * per
        pltpu.sync_copy(x_hbm.at[pl.ds(base, per)], xb_v)
        pltpu.sync_copy(oi_hbm, oi_v)
        pltpu.sync_copy(val_hbm, val_v)
        for t in range(R // L):
            idx = oi_v[pl.ds(t * L, L)]
            v = val_v[pl.ds(t * L, L)]
            loc = idx - base
            m = (loc >= 0) & (loc < per)
            locc = jnp.clip(loc, 0, per - 1)
            plsc.store_scatter(xb_v, [locc], v, mask=m)
        pltpu.sync_copy(xb_v, out_hbm.at[pl.ds(base, per)])

    return k(x, out_idx, vals)


def kernel(x, weights, biases, out_idx, out_sign, max_iters):
    R, S = weights.shape
    wg = jnp.reshape(
        jnp.transpose(
            jnp.reshape(weights, (R // 8, 8, S // 128, 128)), (0, 2, 1, 3)
        ),
        (R * S // 16, 16),
    )
    x16 = jnp.reshape(x, (S // 16, 16))
    cw = 4096
    x2d = jnp.reshape(x, (S // cw, cw))
    base2 = _tc_dense(weights, jnp.transpose(x2d), x2d, cw)
    wsub, xsub = _sc_gather(wg, out_idx, x16, R, S)
    mi = jnp.reshape(jnp.asarray(max_iters, jnp.int32), (1,))
    vfin = _tc_iterate(
        jnp.reshape(base2, (1, R)),
        jnp.reshape(biases, (1, R)),
        jnp.transpose(wsub),
        jnp.reshape(xsub, (1, R)),
        jnp.reshape(out_sign, (1, R)),
        mi,
    )
    return _sc_scatter(x, out_idx, jnp.reshape(vfin, (R,)))
